# baseline (speedup 1.0000x reference)
import functools

import jax
import jax.numpy as jnp
from jax.experimental import pallas as pl
from jax.experimental.pallas import tpu as pltpu

H = 4096
N_TOK = 8192
EPS = 1e-6
FP8_MAX = 448.0

WB = 256
NB = H // WB
BM = 512


def _wprep_body(w1_ref, w2_ref,
                q1_ref, q2_ref, s1_ref, s2_ref, msc_ref):
    p = pl.program_id(0)
    i = pl.program_id(1)

    @pl.when(p == 0)
    def _phase_max():
        for k, w_ref in enumerate((w1_ref, w2_ref)):
            m = jnp.max(jnp.abs(w_ref[...]))
            prev = jnp.where(i == 0, 0.0, msc_ref[k])
            msc_ref[k] = jnp.maximum(prev, m)

    @pl.when(p == 1)
    def _phase_cast():
        for k, (w_ref, q_ref, s_ref) in enumerate((
                (w1_ref, q1_ref, s1_ref),
                (w2_ref, q2_ref, s2_ref))):
            scale = jnp.maximum(msc_ref[k] / FP8_MAX, 1e-12)
            inv = 1.0 / scale
            q_ref[...] = jnp.clip(
                w_ref[...] * inv, -FP8_MAX, FP8_MAX).astype(jnp.float8_e4m3fn)
            s_ref[...] = scale.reshape(1, 1)


def _wprep(W1, W2):
    wspec = pl.BlockSpec((WB, H), lambda p, i: (i, 0))
    qspec = pl.BlockSpec((WB, H), lambda p, i: (p * i, 0))
    sspec = pl.BlockSpec((1, 1), lambda p, i: (0, 0))
    return pl.pallas_call(
        _wprep_body,
        grid=(2, NB),
        in_specs=[wspec, wspec],
        out_specs=[qspec, qspec, sspec, sspec],
        out_shape=[jax.ShapeDtypeStruct((H, H), jnp.float8_e4m3fn)] * 2
        + [jax.ShapeDtypeStruct((1, 1), jnp.float32)] * 2,
        scratch_shapes=[pltpu.SMEM((2,), jnp.float32)],
        compiler_params=pltpu.CompilerParams(
            dimension_semantics=("arbitrary", "arbitrary"),
        ),
        name="wprep",
    )(W1, W2)


def _wcast3_body(w_ref, m_ref, q_ref, s_ref):
    scale = jnp.maximum(m_ref[0, 0] / FP8_MAX, 1e-12)
    inv = 1.0 / scale
    q_ref[...] = jnp.clip(
        w_ref[...] * inv, -FP8_MAX, FP8_MAX).astype(jnp.float8_e4m3fn)
    s_ref[...] = scale.reshape(1, 1)


def _wcast3(W3, m3):
    return pl.pallas_call(
        _wcast3_body,
        grid=(NB,),
        in_specs=[pl.BlockSpec((WB, H), lambda i: (i, 0)),
                  pl.BlockSpec(memory_space=pltpu.SMEM)],
        out_specs=[pl.BlockSpec((WB, H), lambda i: (i, 0)),
                   pl.BlockSpec((1, 1), lambda i: (0, 0))],
        out_shape=[jax.ShapeDtypeStruct((H, H), jnp.float8_e4m3fn),
                   jax.ShapeDtypeStruct((1, 1), jnp.float32)],
        compiler_params=pltpu.CompilerParams(
            dimension_semantics=("parallel",),
        ),
        name="wcast3",
    )(W3, m3)


def _layer_body(do_relu, do_final_norm, *refs):
    resid_ref, nw_ref, qw_ref, sw_ref, out_ref = refs
    r = resid_ref[...]
    if do_relu:
        r = jnp.maximum(r, 0.0)
    t = r * nw_ref[0:1, :]
    var = jnp.mean(r * r, axis=-1, keepdims=True)
    rs = jax.lax.rsqrt(var + EPS)
    amax = rs * jnp.max(jnp.abs(t), axis=-1, keepdims=True)
    s = jnp.maximum(amax / FP8_MAX, 1e-12)
    q = jnp.clip(t * (rs / s), -FP8_MAX, FP8_MAX).astype(jnp.float8_e4m3fn)
    out_ref[...] = jax.lax.dot_general(
        q, qw_ref[...], (((1,), (1,)), ((), ())),
        preferred_element_type=jnp.float32)
    new_resid = out_ref[...] * (s * sw_ref[0, 0]) + r
    if do_final_norm:
        var2 = jnp.mean(new_resid * new_resid, axis=-1, keepdims=True)
        out_ref[...] = new_resid * jax.lax.rsqrt(var2 + EPS) * nw_ref[1:2, :]
    else:
        out_ref[...] = new_resid


def _layer(resid, nw, qw, sw, nwf=None, do_relu=False):
    nwarr = (nw.reshape(1, H) if nwf is None
             else jnp.stack([nw, nwf], axis=0))
    in_specs = [
        pl.BlockSpec((BM, H), lambda i: (i, 0)),
        pl.BlockSpec(nwarr.shape, lambda i: (0, 0)),
        pl.BlockSpec((H, H), lambda i: (0, 0)),
        pl.BlockSpec(memory_space=pltpu.SMEM),
    ]
    args = [resid, nwarr, qw, sw]
    body = functools.partial(_layer_body, do_relu, nwf is not None)
    return pl.pallas_call(
        body,
        grid=(N_TOK // BM,),
        in_specs=in_specs,
        out_specs=pl.BlockSpec((BM, H), lambda i: (i, 0)),
        out_shape=jax.ShapeDtypeStruct((N_TOK, H), jnp.float32),
        compiler_params=pltpu.CompilerParams(
            dimension_semantics=("parallel",),
            vmem_limit_bytes=int(58.5 * 1024 * 1024),
        ),
        name="fused_layer",
    )(*args)


FBM = 256


def _fused2_body(x_ref, nw0_ref, nw1_ref, qw1_ref, qw2_ref, s_ref, w3_ref,
                 out_ref, m3_ref, msc_ref):
    i = pl.program_id(0)
    m = jnp.max(jnp.abs(w3_ref[...]))
    prev = jnp.where(i == 0, 0.0, msc_ref[0])
    cur = jnp.maximum(prev, m)
    msc_ref[0] = cur
    m3_ref[...] = cur.reshape(1, 1)
    r = jnp.maximum(x_ref[...], 0.0)
    for li, (nw_ref, qw_ref) in enumerate(
            ((nw0_ref, qw1_ref), (nw1_ref, qw2_ref))):
        t = (r * nw_ref[...]).astype(jnp.bfloat16)
        var = jnp.mean(r * r, axis=-1, keepdims=True)
        rs = jax.lax.rsqrt(var + EPS)
        amax = rs * jnp.max(jnp.abs(t).astype(jnp.float32), axis=-1,
                            keepdims=True)
        s = jnp.maximum(amax / FP8_MAX, 1e-12)
        q = jnp.clip(t.astype(jnp.float32) * (rs / s),
                     -FP8_MAX, FP8_MAX).astype(jnp.float8_e4m3fn)
        out_ref[...] = jax.lax.dot_general(
            q, qw_ref[...], (((1,), (1,)), ((), ())),
            preferred_element_type=jnp.float32)
        r = out_ref[...] * (s * s_ref[0, li]) + r
    out_ref[...] = r


W3B = H // (N_TOK // FBM)


def _fused2(x, nw0, nw1, qw1, qw2, sws, W3):
    vspec = pl.BlockSpec((1, H), lambda i: (0, 0))
    wspec = pl.BlockSpec((H, H), lambda i: (0, 0))
    return pl.pallas_call(
        _fused2_body,
        grid=(N_TOK // FBM,),
        in_specs=[
            pl.BlockSpec((FBM, H), lambda i: (i, 0)),
            vspec, vspec,
            wspec, wspec,
            pl.BlockSpec(memory_space=pltpu.SMEM),
            pl.BlockSpec((W3B, H), lambda i: (i, 0)),
        ],
        out_specs=[pl.BlockSpec((FBM, H), lambda i: (i, 0)),
                   pl.BlockSpec((1, 1), lambda i: (0, 0))],
        out_shape=[jax.ShapeDtypeStruct((N_TOK, H), jnp.float32),
                   jax.ShapeDtypeStruct((1, 1), jnp.float32)],
        scratch_shapes=[pltpu.SMEM((1,), jnp.float32)],
        compiler_params=pltpu.CompilerParams(
            dimension_semantics=("arbitrary",),
            vmem_limit_bytes=int(58.5 * 1024 * 1024),
        ),
        name="fused2",
    )(x, nw0.reshape(1, H), nw1.reshape(1, H), qw1, qw2, sws, W3)


def kernel(x, nw0, nw1, nw2, nw3, W1, W2, W3):
    qw1, qw2, s1, s2 = _wprep(W1, W2)
    sws = jnp.concatenate([s1, s2], axis=1)
    h2, m3 = _fused2(x, nw0, nw1, qw1, qw2, sws, W3)
    qw3, s3 = _wcast3(W3, m3)
    return _layer(h2, nw2, qw3, s3, nwf=nw3)
